# baseline (speedup 1.0000x reference)
import functools

import jax
import jax.numpy as jnp
from jax import lax
from jax.experimental import pallas as pl
from jax.experimental.pallas import tpu as pltpu
from jax.experimental.pallas import tpu_sc as plsc

GENE_NUM = 100000
D = 32
FW = 160
SW = 4 * FW
SROWS = GENE_NUM // 4
N = 16384
NC = 2
NS = 16
L = 16
NW = NC * NS

G = 128
NGCH = GENE_NUM // G
GTAIL = GENE_NUM - NGCH * G
SR = G // 4

RPW = N // NW
CH = 64
NCH = RPW // CH
ORPW = RPW // 2


def _tr_body(weT, beT, bmT, wmT, tail, scr, stage_v, outb_v, isem, osem):
    wid = lax.axis_index("s") * NC + lax.axis_index("c")
    rem = NGCH - (NGCH // NW) * NW
    nch = jnp.where(wid < rem, NGCH // NW + 1, NGCH // NW)

    tables = ((weT, 0, 4), (beT, 32, 4), (bmT, 64, 4), (wmT, 96, 8))

    def fire(goff, width, p):
        cps = []
        for t, rbase, nrb in tables:
            for rb in range(nrb):
                cps.append(pltpu.async_copy(
                    t.at[pl.ds(rb * 8, 8), pl.ds(goff, width)],
                    stage_v.at[pl.ds(rbase + rb * 8, 8), pl.ds(0, width)],
                    isem.at[p]))
        return cps

    def transpose_genes(p, ngen):
        nk = (4 * 2 * D) // L

        def gloop(g, carry):
            q = jnp.bitwise_and(g, 3)
            r = lax.shift_right_logical(g, 2)
            cb = q * FW
            col = jnp.full((L,), g, jnp.int32)
            for k in range(nk):
                rik = k * L + lax.iota(jnp.int32, L)
                v = plsc.load_gather(stage_v, [rik, col])
                outb_v[r, pl.ds(pl.multiple_of(cb + k * L, L), L)] = v
            return carry
        lax.fori_loop(0, ngen, gloop, 0, unroll=4)

    def body(j, carry):
        c = wid + j * NW
        p = 0
        for cp in fire(c * G, G, p):
            cp.wait()
        transpose_genes(p, G)
        pltpu.sync_copy(outb_v, scr.at[pl.ds(c * SR, SR)])
        return carry
    lax.fori_loop(0, nch, body, 0)

    @pl.when(wid == NW - 1)
    def _():
        pltpu.sync_copy(tail, scr.at[pl.ds(NGCH * SR, GTAIL // 4)])


_tr_kernel = functools.partial(
    pl.kernel,
    mesh=plsc.VectorSubcoreMesh(core_axis_name="c", subcore_axis_name="s"),
    out_type=jax.ShapeDtypeStruct((SROWS, SW), jnp.float32),
    scratch_types=[
        pltpu.VMEM((4 * 2 * D, G + 1), jnp.float32),
        pltpu.VMEM((SR, SW), jnp.float32),
        pltpu.SemaphoreType.DMA((2,)),
        pltpu.SemaphoreType.DMA((2,)),
    ],
    compiler_params=pltpu.CompilerParams(needs_layout_passes=False),
)(_tr_body)


def _ga_body(tab_hbm, pos_hbm, exp_hbm, flg_hbm, out_hbm,
             pos_v, exp_v, flg_v, row_v, qb_v, om_v, tb_v, out_v, sems):
    wid = lax.axis_index("s") * NC + lax.axis_index("c")
    base = wid * RPW

    pltpu.sync_copy(pos_hbm.at[pl.ds(base, RPW)], pos_v)
    pltpu.sync_copy(exp_hbm.at[pl.ds(base, RPW)], exp_v)
    pltpu.sync_copy(flg_hbm.at[pl.ds(base, RPW)], flg_v)

    def mk_idx(i, carry):
        sl = pl.ds(i * L, L)
        p = pos_v[sl]
        qb = lax.shift_left(jnp.bitwise_and(p, 3), 5) * 5
        row_v[sl] = lax.shift_right_logical(p, 2)
        qb_v[sl] = qb
        om_v[sl] = qb + lax.shift_left(flg_v[sl].astype(jnp.int32), 5) + 3 * D
        return carry
    lax.fori_loop(0, RPW // L, mk_idx, 0)

    def fire(c):
        p = c % 2
        s = pl.ds(c * CH, CH)
        return pltpu.async_copy(tab_hbm.at[row_v.at[s]], tb_v.at[p],
                                sems.at[p])

    def compute(c):
        p = c % 2
        cbase = c * CH

        def grp(g, carry):
            r0 = g * L
            sl = pl.ds(cbase + r0, L)
            ev = exp_v[sl]
            qv = qb_v[sl]
            ov = om_v[sl]
            ob = lax.shift_right_logical(cbase + r0, 1)
            for j in range(L):
                r = r0 + j
                e = ev[j]
                qb = pl.multiple_of(qv[j], L)
                om = pl.multiple_of(ov[j], L)
                o2 = ob + j // 2
                par = (j % 2) * (4 * L)
                out_v[o2, pl.ds(par, L)] = (
                    tb_v[p, r, pl.ds(qb, L)] * e
                    + tb_v[p, r, pl.ds(qb + 2 * L, L)])
                out_v[o2, pl.ds(par + L, L)] = (
                    tb_v[p, r, pl.ds(qb + L, L)] * e
                    + tb_v[p, r, pl.ds(qb + 3 * L, L)])
                out_v[o2, pl.ds(par + 2 * L, L)] = (
                    tb_v[p, r, pl.ds(om, L)]
                    + tb_v[p, r, pl.ds(qb + 4 * L, L)])
                out_v[o2, pl.ds(par + 3 * L, L)] = (
                    tb_v[p, r, pl.ds(om + L, L)]
                    + tb_v[p, r, pl.ds(qb + 5 * L, L)])
            return carry
        lax.fori_loop(0, CH // L, grp, 0)

    pending = fire(0)
    for c in range(NCH):
        cur = pending
        if c + 1 < NCH:
            pending = fire(c + 1)
        cur.wait()
        compute(c)

    pltpu.sync_copy(out_v, out_hbm.at[pl.ds(wid * ORPW, ORPW)])


_ga_kernel = functools.partial(
    pl.kernel,
    mesh=plsc.VectorSubcoreMesh(core_axis_name="c", subcore_axis_name="s"),
    out_type=jax.ShapeDtypeStruct((N // 2, 4 * D), jnp.float32),
    scratch_types=[
        pltpu.VMEM((RPW,), jnp.int32),
        pltpu.VMEM((RPW,), jnp.float32),
        pltpu.VMEM((RPW,), jnp.float32),
        pltpu.VMEM((RPW,), jnp.int32),
        pltpu.VMEM((RPW,), jnp.int32),
        pltpu.VMEM((RPW,), jnp.int32),
        pltpu.VMEM((2, CH, SW), jnp.float32),
        pltpu.VMEM((ORPW, 4 * D), jnp.float32),
        pltpu.SemaphoreType.DMA((2,)),
    ],
)(_ga_body)


def kernel(x, pos, weight_exp, bias_exp, weight_mu, bias_mu):
    pos32 = pos.astype(jnp.int32)
    exp_col = x[:, 0]
    flg_col = x[:, 1]
    weT = weight_exp.T
    beT = bias_exp.T
    bmT = bias_mu.T
    wmT = weight_mu.transpose(1, 2, 0).reshape(2 * D, GENE_NUM)
    tg = NGCH * G
    tail = jnp.concatenate(
        [weight_exp[tg:], bias_exp[tg:], bias_mu[tg:],
         weight_mu[tg:].reshape(GTAIL, 2 * D)], axis=1).reshape(
             GTAIL // 4, SW)
    scr = _tr_kernel(weT, beT, bmT, wmT, tail)
    out = _ga_kernel(scr, pos32, exp_col, flg_col)
    return out.reshape(N, 2 * D)


# speedup vs baseline: 2.2849x; 2.2849x over previous
import functools

import jax
import jax.numpy as jnp
from jax import lax
from jax.experimental import pallas as pl
from jax.experimental.pallas import tpu as pltpu
from jax.experimental.pallas import tpu_sc as plsc

GENE_NUM = 100000
D = 32
W = 160
N = 16384
NC = 2
NS = 16
L = 16
NW = NC * NS
RPW = N // NW
CH = 64
NCH = RPW // CH
ORPW = RPW // 2


def _sc_body(tab_hbm, pos_hbm, exp_hbm, flg_hbm, out_hbm,
             pos_v, exp_v, flg_v, off_v, tb_v, out_v, sems):
    wid = lax.axis_index("s") * NC + lax.axis_index("c")
    base = wid * RPW

    pltpu.sync_copy(pos_hbm.at[pl.ds(base, RPW)], pos_v)
    pltpu.sync_copy(exp_hbm.at[pl.ds(base, RPW)], exp_v)
    pltpu.sync_copy(flg_hbm.at[pl.ds(base, RPW)], flg_v)

    def mk_off(i, carry):
        sl = pl.ds(i * L, L)
        off_v[sl] = lax.shift_left(flg_v[sl].astype(jnp.int32), 5) + 3 * D
        return carry
    lax.fori_loop(0, RPW // L, mk_off, 0)

    def fire(c):
        p = c % 2
        s = pl.ds(c * CH, CH)
        return pltpu.async_copy(tab_hbm.at[pos_v.at[s]], tb_v.at[p],
                                sems.at[p])

    def compute(c):
        p = c % 2
        cbase = c * CH

        def grp(g, carry):
            r0 = g * L
            sl = pl.ds(cbase + r0, L)
            ev = exp_v[sl]
            ov = off_v[sl]
            ob = lax.shift_right_logical(cbase + r0, 1)
            for j in range(L):
                r = r0 + j
                e = ev[j]
                om = pl.multiple_of(ov[j], L)
                o2 = ob + j // 2
                par = (j % 2) * (4 * L)
                out_v[o2, pl.ds(par, L)] = (tb_v[p, r, pl.ds(0, L)] * e
                                            + tb_v[p, r, pl.ds(2 * L, L)])
                out_v[o2, pl.ds(par + L, L)] = (tb_v[p, r, pl.ds(L, L)] * e
                                                + tb_v[p, r, pl.ds(3 * L, L)])
                out_v[o2, pl.ds(par + 2 * L, L)] = (tb_v[p, r, pl.ds(om, L)]
                                                    + tb_v[p, r, pl.ds(4 * L, L)])
                out_v[o2, pl.ds(par + 3 * L, L)] = (tb_v[p, r, pl.ds(om + L, L)]
                                                    + tb_v[p, r, pl.ds(5 * L, L)])
            return carry
        lax.fori_loop(0, CH // L, grp, 0)

    pending = fire(0)
    for c in range(NCH):
        cur = pending
        if c + 1 < NCH:
            pending = fire(c + 1)
        cur.wait()
        compute(c)

    pltpu.sync_copy(out_v, out_hbm.at[pl.ds(wid * ORPW, ORPW)])


_sc_kernel = functools.partial(
    pl.kernel,
    mesh=plsc.VectorSubcoreMesh(core_axis_name="c", subcore_axis_name="s"),
    out_type=jax.ShapeDtypeStruct((N // 2, 4 * D), jnp.float32),
    scratch_types=[
        pltpu.VMEM((RPW,), jnp.int32),
        pltpu.VMEM((RPW,), jnp.float32),
        pltpu.VMEM((RPW,), jnp.float32),
        pltpu.VMEM((RPW,), jnp.int32),
        pltpu.VMEM((2, CH, W), jnp.float32),
        pltpu.VMEM((ORPW, 4 * D), jnp.float32),
        pltpu.SemaphoreType.DMA((2,)),
    ],
    compiler_params=pltpu.CompilerParams(use_tc_tiling_on_sc=False),
)(_sc_body)


def kernel(x, pos, weight_exp, bias_exp, weight_mu, bias_mu):
    pos32 = pos.astype(jnp.int32)
    exp_col = x[:, 0]
    flg_col = x[:, 1]
    catT = jnp.concatenate(
        [weight_exp.T, bias_exp.T, bias_mu.T,
         weight_mu.transpose(1, 2, 0).reshape(2 * D, GENE_NUM)], axis=0)
    fused = catT.T
    out = _sc_kernel(fused, pos32, exp_col, flg_col)
    return out.reshape(N, 2 * D)


# speedup vs baseline: 2.8921x; 1.2658x over previous
import functools

import jax
import jax.numpy as jnp
from jax import lax
from jax.experimental import pallas as pl
from jax.experimental.pallas import tpu as pltpu
from jax.experimental.pallas import tpu_sc as plsc

GENE_NUM = 100000
D = 32
N = 16384
NC = 2
NS = 16
L = 16
NW = NC * NS
RPW = N // NW
CH = 128
NCH = RPW // CH


def _sc_body(we_hbm, be_hbm, wm_hbm, bm_hbm, pos_hbm, exp_hbm, flg_hbm,
             out_hbm,
             pos_v, exp_v, flg_v, idx2_v, we_v, be_v, wm_v, bm_v, out_v,
             sem0, sem1, sem2, sem3):
    wid = lax.axis_index("s") * NC + lax.axis_index("c")
    base = wid * RPW

    pltpu.sync_copy(pos_hbm.at[pl.ds(base, RPW)], pos_v)
    pltpu.sync_copy(flg_hbm.at[pl.ds(base, RPW)], flg_v)
    pltpu.sync_copy(exp_hbm.at[pl.ds(base, RPW)], exp_v)

    def mk_idx(i, carry):
        sl = pl.ds(i * L, L)
        idx2_v[sl] = pos_v[sl] * 2 + flg_v[sl].astype(jnp.int32)
        return carry
    lax.fori_loop(0, RPW // L, mk_idx, 0)

    copies = []
    for c in range(NCH):
        s = pl.ds(c * CH, CH)
        copies.append(pltpu.async_copy(we_hbm.at[pos_v.at[s]], we_v.at[s], sem0))
        copies.append(pltpu.async_copy(be_hbm.at[pos_v.at[s]], be_v.at[s], sem1))
        copies.append(pltpu.async_copy(wm_hbm.at[idx2_v.at[s]], wm_v.at[s], sem2))
        copies.append(pltpu.async_copy(bm_hbm.at[pos_v.at[s]], bm_v.at[s], sem3))
    for cp in copies:
        cp.wait()

    h0, h1 = pl.ds(0, L), pl.ds(L, L)

    def grp(g, carry):
        ev = exp_v[pl.ds(g * L, L)]
        for j in range(L):
            r = g * L + j
            e = ev[j]
            out_v[r, h0] = we_v[r, h0] * e + be_v[r, h0]
            out_v[r, h1] = we_v[r, h1] * e + be_v[r, h1]
            out_v[r, pl.ds(2 * L, L)] = wm_v[r, h0] + bm_v[r, h0]
            out_v[r, pl.ds(3 * L, L)] = wm_v[r, h1] + bm_v[r, h1]
        return carry
    lax.fori_loop(0, RPW // L, grp, 0)

    pltpu.sync_copy(out_v, out_hbm.at[pl.ds(base, RPW)])


_sc_kernel = functools.partial(
    pl.kernel,
    mesh=plsc.VectorSubcoreMesh(core_axis_name="c", subcore_axis_name="s"),
    out_type=jax.ShapeDtypeStruct((N, 2 * D), jnp.float32),
    scratch_types=[
        pltpu.VMEM((RPW,), jnp.int32),
        pltpu.VMEM((RPW,), jnp.float32),
        pltpu.VMEM((RPW,), jnp.float32),
        pltpu.VMEM((RPW,), jnp.int32),
        pltpu.VMEM((RPW, D), jnp.float32),
        pltpu.VMEM((RPW, D), jnp.float32),
        pltpu.VMEM((RPW, D), jnp.float32),
        pltpu.VMEM((RPW, D), jnp.float32),
        pltpu.VMEM((RPW, 2 * D), jnp.float32),
        pltpu.SemaphoreType.DMA,
        pltpu.SemaphoreType.DMA,
        pltpu.SemaphoreType.DMA,
        pltpu.SemaphoreType.DMA,
    ],
    compiler_params=pltpu.CompilerParams(use_tc_tiling_on_sc=False),
)(_sc_body)


def kernel(x, pos, weight_exp, bias_exp, weight_mu, bias_mu):
    pos32 = pos.astype(jnp.int32)
    exp_col = x[:, 0]
    flg_col = x[:, 1]
    wm2 = weight_mu.reshape(2 * GENE_NUM, D)
    return _sc_kernel(weight_exp, bias_exp, wm2, bias_mu, pos32,
                      exp_col, flg_col)
